# Initial kernel scaffold; baseline (speedup 1.0000x reference)
import jax
import jax.numpy as jnp
from jax.experimental import pallas as pl


def kernel(x):
    raise NotImplementedError("write your pallas kernel here")


# speedup vs baseline: 1.7110x; 1.7110x over previous
import jax
import jax.numpy as jnp
from jax.experimental import pallas as pl


def _body(x_ref, o_ref):
    o_ref[...] = x_ref[0, 2:4, :]


def kernel(x):
    return pl.pallas_call(
        _body,
        out_shape=jax.ShapeDtypeStruct((2, 128), jnp.float32),
        grid=(1,),
        in_specs=[pl.BlockSpec((1, 8, 128), lambda i: (1, 0, 0))],
        out_specs=pl.BlockSpec((2, 128), lambda i: (0, 0)),
    )(x)
